# Initial kernel scaffold; baseline (speedup 1.0000x reference)
import jax
import jax.numpy as jnp
from jax.experimental import pallas as pl


def kernel(pred, gt, pcd_radius):
    raise NotImplementedError("write your pallas kernel here")


# speedup vs baseline: 60.3668x; 60.3668x over previous
import functools

import jax
import jax.numpy as jnp
from jax.experimental import pallas as pl

ALPHA = 1.0
BETA = 1.0
NN_SIZE = 5
RADIUS = 0.07
H2 = 0.03 * 0.03
EPS = 1e-12
FX, FY = 111, 62
SIGMA_INV = 100.0
B, N = 4, 1024
P = B * N


def _sqdist_tile(a_cols, b_rows):
    acc = None
    for c in range(3):
        d = a_cols[:, c : c + 1] - b_rows[c : c + 1, :]
        t = d * d
        acc = t if acc is None else acc + t
    return acc


def _loss_kernel(pred_c, pred_r, gt_c, gt_r, pxy, gxy, rad,
                 cd_out, rep_out, f_out):
    col_iota = jax.lax.broadcasted_iota(jnp.int32, (N, N), 1)
    inf = jnp.float32(jnp.inf)
    big = jnp.int32(2 ** 30)

    cd_sum = jnp.float32(0.0)
    rep_sum = jnp.float32(0.0)
    for b in range(B):
        pc = pred_c[b]
        pr = pred_r[b]
        gc = gt_c[b]

        dgp = _sqdist_tile(gc, pr)
        cost_for = jnp.min(dgp, axis=1, keepdims=True)
        cost_bac = jnp.min(dgp, axis=0, keepdims=True)
        bsum = 0.8 * jnp.sum(cost_for) + 0.2 * jnp.sum(cost_bac)
        cd_sum = cd_sum + bsum / rad[b, 0]

        dpp = _sqdist_tile(pc, pr)
        for k in range(NN_SIZE):
            m = jnp.min(dpp, axis=1, keepdims=True)
            if k > 0:
                d2 = jnp.maximum(m, EPS)
                dist = jnp.sqrt(d2)
                w = jnp.exp(-d2 / H2)
                rep_sum = rep_sum + jnp.sum((RADIUS - dist) * w)
            if k < NN_SIZE - 1:
                cand = jnp.where(dpp == m, col_iota, big)
                jmin = jnp.min(cand, axis=1, keepdims=True)
                dpp = jnp.where(col_iota == jmin, inf, dpp)

    cd_out[:, :] = jnp.reshape(100.0 * cd_sum / (B * N), (1, 1))
    rep_out[:, :] = jnp.reshape(ALPHA * rep_sum / (B * N * (NN_SIZE - 1)), (1, 1))

    row2 = jax.lax.broadcasted_iota(jnp.int32, (2, 1), 0)
    scale = jnp.where(row2 == 0, FX - 1.0, FY - 1.0).astype(jnp.float32)

    def norm_xy(xy):
        mn = jnp.min(xy, axis=1, keepdims=True)
        sh = xy - mn
        mx = jnp.max(sh, axis=1, keepdims=True)
        return sh * (scale / mx)

    gxg = jax.lax.broadcasted_iota(jnp.int32, (128, 1), 0).astype(jnp.float32)

    def gauss_tables(xy):
        nxy = norm_xy(xy)
        dx = gxg - nxy[0:1, :]
        dy = gxg - nxy[1:2, :]
        return jnp.exp(dx * dx * (-SIGMA_INV)), jnp.exp(dy * dy * (-SIGMA_INV))

    xp, yp = gauss_tables(pxy[...])
    xg, yg = gauss_tables(gxy[...])
    a = jnp.concatenate([xp, xg], axis=1)
    bm = jnp.concatenate([yp, -yg], axis=1)
    diff = jax.lax.dot_general(a, bm, (((1,), (1,)), ((), ())),
                               preferred_element_type=jnp.float32)
    rmask = jax.lax.broadcasted_iota(jnp.int32, (128, 128), 0) < FX
    cmask = jax.lax.broadcasted_iota(jnp.int32, (128, 128), 1) < FY
    diff = jnp.where(rmask & cmask, diff, 0.0)
    f_out[:, :] = jnp.reshape(BETA * jnp.sum(diff * diff) / (FX * FY), (1, 1))


@functools.partial(jax.jit, static_argnames=())
def kernel(pred, gt, pcd_radius):
    pred = pred.astype(jnp.float32)
    gt = gt.astype(jnp.float32)
    pred_r = jnp.transpose(pred, (0, 2, 1))
    gt_r = jnp.transpose(gt, (0, 2, 1))
    pxy = pred[..., 1:3].reshape(P, 2).T
    gxy = gt[..., 1:3].reshape(P, 2).T

    out = pl.pallas_call(
        _loss_kernel,
        out_shape=(
            jax.ShapeDtypeStruct((1, 1), jnp.float32),
            jax.ShapeDtypeStruct((1, 1), jnp.float32),
            jax.ShapeDtypeStruct((1, 1), jnp.float32),
        ),
    )(pred, pred_r, gt, gt_r, pxy, gxy, pcd_radius.astype(jnp.float32))
    cd, rep, fl = out
    return (cd[0, 0], rep[0, 0], fl[0, 0])


# speedup vs baseline: 62.0531x; 1.0279x over previous
import functools

import jax
import jax.numpy as jnp
from jax.experimental import pallas as pl
from jax.experimental.pallas import tpu as pltpu

ALPHA = 1.0
BETA = 1.0
NN_SIZE = 5
RADIUS = 0.07
H2 = 0.03 * 0.03
EPS = 1e-12
FX, FY = 111, 62
SIGMA_INV = 100.0
B, N = 4, 1024
P = B * N
BPI = B // 2
PH = P // 2
CD_ROW = 112
REP_ROW = 113


def _sqdist_tile(a_cols, b_rows):
    acc = None
    for c in range(3):
        d = a_cols[:, c : c + 1] - b_rows[c : c + 1, :]
        t = d * d
        acc = t if acc is None else acc + t
    return acc


def _loss_kernel(pred_c, pred_r, gt_c, gt_r, pxy, gxy, out):
    i = pl.program_id(0)
    col_iota = jax.lax.broadcasted_iota(jnp.int32, (N, N), 1)
    row_iota = jax.lax.broadcasted_iota(jnp.int32, (N, N), 0)
    inf = jnp.float32(jnp.inf)
    big = jnp.int32(2 ** 30)

    cd_bsum = []
    rep_sum = jnp.float32(0.0)
    for b in range(BPI):
        pc = pred_c[b]
        pr = pred_r[b]
        gc = gt_c[b]

        dgp = _sqdist_tile(gc, pr)
        cost_for = jnp.min(dgp, axis=1, keepdims=True)
        cost_bac = jnp.min(dgp, axis=0, keepdims=True)
        cd_bsum.append(0.8 * jnp.sum(cost_for) + 0.2 * jnp.sum(cost_bac))

        dpp = _sqdist_tile(pc, pr)
        dpp = jnp.where(col_iota == row_iota, inf, dpp)
        for k in range(NN_SIZE - 1):
            m = jnp.min(dpp, axis=1, keepdims=True)
            d2 = jnp.maximum(m, EPS)
            dist = jnp.sqrt(d2)
            w = jnp.exp(-d2 / H2)
            rep_sum = rep_sum + jnp.sum((RADIUS - dist) * w)
            if k < NN_SIZE - 2:
                cand = jnp.where(dpp == m, col_iota, big)
                jmin = jnp.min(cand, axis=1, keepdims=True)
                dpp = jnp.where(col_iota == jmin, inf, dpp)

    row2 = jax.lax.broadcasted_iota(jnp.int32, (2, 1), 0)
    scale = jnp.where(row2 == 0, FX - 1.0, FY - 1.0).astype(jnp.float32)
    gxg = jax.lax.broadcasted_iota(jnp.int32, (128, 1), 0).astype(jnp.float32)

    def gauss_tables(xy_ref):
        xy = xy_ref[...]
        mn = jnp.min(xy, axis=1, keepdims=True)
        mx = jnp.max(xy - mn, axis=1, keepdims=True)
        half = xy_ref[:, pl.ds(i * PH, PH)]
        nxy = (half - mn) * (scale / mx)
        dx = gxg - nxy[0:1, :]
        dy = gxg - nxy[1:2, :]
        return jnp.exp(dx * dx * (-SIGMA_INV)), jnp.exp(dy * dy * (-SIGMA_INV))

    xp, yp = gauss_tables(pxy)
    xg, yg = gauss_tables(gxy)
    a = jnp.concatenate([xp, xg], axis=1)
    bm = jnp.concatenate([yp, -yg], axis=1)
    part = jax.lax.dot_general(a, bm, (((1,), (1,)), ((), ())),
                               preferred_element_type=jnp.float32)
    r128 = jax.lax.broadcasted_iota(jnp.int32, (128, 128), 0)
    c128 = jax.lax.broadcasted_iota(jnp.int32, (128, 128), 1)
    tile = jnp.where((r128 < FX) & (c128 < FY), part, 0.0)
    tile = jnp.where((r128 == CD_ROW) & (c128 == 0), cd_bsum[0], tile)
    tile = jnp.where((r128 == CD_ROW) & (c128 == 1), cd_bsum[1], tile)
    tile = jnp.where((r128 == REP_ROW) & (c128 == 0), rep_sum, tile)
    out[0, :, :] = tile


@functools.partial(jax.jit, static_argnames=())
def kernel(pred, gt, pcd_radius):
    pred = pred.astype(jnp.float32)
    gt = gt.astype(jnp.float32)
    pred_r = jnp.transpose(pred, (0, 2, 1))
    gt_r = jnp.transpose(gt, (0, 2, 1))
    pxy = pred[..., 1:3].reshape(P, 2).T
    gxy = gt[..., 1:3].reshape(P, 2).T

    tiles = pl.pallas_call(
        _loss_kernel,
        grid=(2,),
        in_specs=[
            pl.BlockSpec((BPI, N, 3), lambda i: (i, 0, 0)),
            pl.BlockSpec((BPI, 3, N), lambda i: (i, 0, 0)),
            pl.BlockSpec((BPI, N, 3), lambda i: (i, 0, 0)),
            pl.BlockSpec((BPI, 3, N), lambda i: (i, 0, 0)),
            pl.BlockSpec((2, P), lambda i: (0, 0)),
            pl.BlockSpec((2, P), lambda i: (0, 0)),
        ],
        out_specs=pl.BlockSpec((1, 128, 128), lambda i: (i, 0, 0)),
        out_shape=jax.ShapeDtypeStruct((2, 128, 128), jnp.float32),
        compiler_params=pltpu.CompilerParams(
            dimension_semantics=("parallel",),
        ),
    )(pred, pred_r, gt, gt_r, pxy, gxy)

    rad = pcd_radius.astype(jnp.float32)[:, 0]
    cd_bsums = tiles[:, CD_ROW, 0:BPI].reshape(B)
    cd = 100.0 * jnp.sum(cd_bsums / rad) / (B * N)
    rep = ALPHA * (tiles[0, REP_ROW, 0] + tiles[1, REP_ROW, 0]) / (
        B * N * (NN_SIZE - 1))
    fg = tiles[0, :FX, :FY] + tiles[1, :FX, :FY]
    fl = BETA * jnp.sum(fg * fg) / (FX * FY)
    return (cd, rep, fl)
